# baseline (speedup 1.0000x reference)
import functools

import jax
import jax.numpy as jnp
from jax import lax
from jax.experimental import pallas as pl
from jax.experimental.pallas import tpu as pltpu
from jax.experimental.pallas import tpu_sc as plsc

EPS = 1e-5
LANES = 16
NC, NS = 2, 16
NW = NC * NS
U = 128


def _tc_pre(x, w1i, w1j):
    n, d = x.shape

    def body(x_ref, wi_ref, wj_ref, p_ref, q_ref):
        xv = x_ref[...]
        p_ref[...] = jnp.dot(xv, wi_ref[...], preferred_element_type=jnp.float32)
        q_ref[...] = jnp.dot(xv, wj_ref[...], preferred_element_type=jnp.float32)

    return pl.pallas_call(
        body,
        out_shape=(jax.ShapeDtypeStruct((n, d), jnp.float32),
                   jax.ShapeDtypeStruct((n, d), jnp.float32)),
    )(x, w1i, w1j)


def _sc_gather(p_tab, q_tab, dst3d, src3d, nu_real):
    _, upt, u = dst3d.shape
    n, d = p_tab.shape
    e = nu_real * u
    mesh = plsc.VectorSubcoreMesh(core_axis_name="c", subcore_axis_name="s")

    @functools.partial(
        pl.kernel,
        out_type=jax.ShapeDtypeStruct((e, d), jnp.float32),
        mesh=mesh,
        scratch_types=[
            pltpu.VMEM((upt, u), jnp.int32),
            pltpu.VMEM((upt, u), jnp.int32),
            pltpu.VMEM((2, u, d), jnp.float32),
            pltpu.VMEM((2, u, d), jnp.float32),
            pltpu.SemaphoreType.DMA,
            pltpu.SemaphoreType.DMA,
        ],
    )
    def k(p_hbm, q_hbm, d_hbm, s_hbm, out_hbm, idxd, idxs, bufa, bufb, gsem,
          wsem):
        w = lax.axis_index("s") * NC + lax.axis_index("c")
        u0 = w * upt
        nu = jnp.clip(nu_real - u0, 0, upt)
        pltpu.sync_copy(d_hbm.at[w], idxd)
        pltpu.sync_copy(s_hbm.at[w], idxs)

        def start_gather(j, slot):
            pltpu.async_copy(p_hbm.at[idxd.at[j]], bufa.at[slot], gsem)
            pltpu.async_copy(q_hbm.at[idxs.at[j]], bufb.at[slot], gsem)

        def drain(desc_src, desc_dst, sem):
            pltpu.make_async_copy(desc_src, desc_dst, sem).wait()

        @pl.when(nu > 0)
        def _():
            start_gather(0, 0)

        def unit(j, carry):
            slot = j % 2
            nslot = 1 - slot

            @pl.when((j + 1 < nu) & (j >= 1))
            def _():
                drain(bufa.at[nslot], out_hbm.at[pl.ds(0, u)], wsem)

            @pl.when(j + 1 < nu)
            def _():
                start_gather(j + 1, nslot)

            drain(p_hbm.at[idxd.at[j]], bufa.at[slot], gsem)
            drain(q_hbm.at[idxs.at[j]], bufb.at[slot], gsem)

            def row(i, c2):
                for t in range(d // LANES):
                    sl = pl.ds(t * LANES, LANES)
                    bufa[slot, i, sl] = bufa[slot, i, sl] + bufb[slot, i, sl]
                return c2

            lax.fori_loop(0, u, row, 0)
            pltpu.async_copy(bufa.at[slot], out_hbm.at[pl.ds((u0 + j) * u, u)],
                             wsem)
            return carry

        lax.fori_loop(0, nu, unit, 0)

        @pl.when(nu >= 2)
        def _():
            drain(bufa.at[0], out_hbm.at[pl.ds(0, u)], wsem)

        @pl.when(nu >= 1)
        def _():
            drain(bufa.at[0], out_hbm.at[pl.ds(0, u)], wsem)

    return k(p_tab, q_tab, dst3d, src3d)


def _sc_scatter(ue, dst3d, nu_real, n, npad):
    e, d = ue.shape
    _, upt, u = dst3d.shape
    rpt = npad // NS
    mesh = plsc.VectorSubcoreMesh(core_axis_name="c", subcore_axis_name="s")

    @functools.partial(
        pl.kernel,
        out_type=jax.ShapeDtypeStruct((NC, npad, d), jnp.float32),
        mesh=mesh,
        scratch_types=[
            pltpu.VMEM((upt, u), jnp.int32),
            pltpu.VMEM((2, u, d), jnp.float32),
            pltpu.VMEM_SHARED((npad, d), jnp.float32),
            pltpu.SemaphoreType.DMA,
            pltpu.SemaphoreType.DMA,
        ],
    )
    def k(ue_hbm, d_hbm, out_hbm, idxd, buf, agg_sh, rsem, ssem):
        c = lax.axis_index("c")
        s = lax.axis_index("s")
        w = s * NC + c
        u0 = w * upt
        nu = jnp.clip(nu_real - u0, 0, upt)

        def zrow(i, c2):
            for t in range(d // LANES):
                buf[0, i, pl.ds(t * LANES, LANES)] = jnp.zeros((LANES,),
                                                               jnp.float32)
            return c2

        lax.fori_loop(0, u, zrow, 0)
        nz, rem = divmod(rpt, u)
        for z in range(nz):
            pltpu.sync_copy(buf.at[0], agg_sh.at[pl.ds(s * rpt + z * u, u)])
        if rem:
            pltpu.sync_copy(buf.at[0, pl.ds(0, rem)],
                            agg_sh.at[pl.ds(s * rpt + nz * u, rem)])
        pltpu.sync_copy(d_hbm.at[w], idxd)
        plsc.subcore_barrier()

        def drain(desc_src, desc_dst, sem, add=False):
            pltpu.make_async_copy(desc_src, desc_dst, sem).wait()

        @pl.when(nu > 0)
        def _():
            pltpu.async_copy(ue_hbm.at[pl.ds(u0 * u, u)], buf.at[0], rsem)

        def unit(j, carry):
            slot = j % 2
            nslot = 1 - slot

            @pl.when((j + 1 < nu) & (j >= 1))
            def _():
                drain(buf.at[nslot], agg_sh.at[idxd.at[j]], ssem)

            @pl.when(j + 1 < nu)
            def _():
                pltpu.async_copy(ue_hbm.at[pl.ds((u0 + j + 1) * u, u)],
                                 buf.at[nslot], rsem)

            drain(ue_hbm.at[pl.ds(u0 * u, u)], buf.at[slot], rsem)
            pltpu.async_copy(buf.at[slot], agg_sh.at[idxd.at[j]], ssem,
                             add=True)
            return carry

        lax.fori_loop(0, nu, unit, 0)

        @pl.when(nu >= 2)
        def _():
            drain(buf.at[0], agg_sh.at[idxd.at[0]], ssem)

        @pl.when(nu >= 1)
        def _():
            drain(buf.at[0], agg_sh.at[idxd.at[0]], ssem)

        plsc.subcore_barrier()
        pltpu.sync_copy(agg_sh.at[pl.ds(s * rpt, rpt)],
                        out_hbm.at[c, pl.ds(s * rpt, rpt)])

    return k(ue, dst3d)


def _tc_stats(g, a, w1e, block):
    e, d = g.shape
    nb = e // block

    def body(g_ref, a_ref, w_ref, out_ref, acc_ref):
        i = pl.program_id(0)
        h = g_ref[...] + jnp.dot(a_ref[...], w_ref[...],
                                 preferred_element_type=jnp.float32)
        blk = jnp.concatenate(
            [jnp.sum(h, axis=0, keepdims=True),
             jnp.sum(h * h, axis=0, keepdims=True)], axis=0)

        @pl.when(i == 0)
        def _():
            acc_ref[...] = blk

        @pl.when(i > 0)
        def _():
            acc_ref[...] = acc_ref[...] + blk

        @pl.when(i == nb - 1)
        def _():
            out_ref[...] = acc_ref[...]

    return pl.pallas_call(
        body,
        grid=(nb,),
        in_specs=[pl.BlockSpec((block, d), lambda i: (i, 0)),
                  pl.BlockSpec((block, d), lambda i: (i, 0)),
                  pl.BlockSpec((d, d), lambda i: (0, 0))],
        out_specs=pl.BlockSpec((2, d), lambda i: (0, 0)),
        out_shape=jax.ShapeDtypeStruct((2, d), jnp.float32),
        scratch_shapes=[pltpu.VMEM((2, d), jnp.float32)],
    )(g, a, w1e)


def _tc_ue(g, a, stats, w1e, b1, gamma, beta, w2, b2, block):
    e, d = g.shape
    nb = e // block
    ef = float(e)

    def body(g_ref, a_ref, st_ref, w1_ref, b1_ref, ga_ref, be_ref, w2_ref,
             b2_ref, out_ref):
        mean = st_ref[0:1, :] / ef
        var = st_ref[1:2, :] / ef - mean * mean
        mu = mean + b1_ref[...]
        scale = lax.rsqrt(var + EPS) * ga_ref[...]
        av = a_ref[...]
        h = g_ref[...] + jnp.dot(av, w1_ref[...],
                                 preferred_element_type=jnp.float32) + b1_ref[...]
        h = (h - mu) * scale + be_ref[...]
        h = jnp.maximum(h, 0.0)
        h = jnp.dot(h, w2_ref[...], preferred_element_type=jnp.float32) + b2_ref[...]
        out_ref[...] = jnp.maximum(h, 0.0) + av

    return pl.pallas_call(
        body,
        grid=(nb,),
        in_specs=[pl.BlockSpec((block, d), lambda i: (i, 0)),
                  pl.BlockSpec((block, d), lambda i: (i, 0)),
                  pl.BlockSpec((2, d), lambda i: (0, 0)),
                  pl.BlockSpec((d, d), lambda i: (0, 0)),
                  pl.BlockSpec((1, d), lambda i: (0, 0)),
                  pl.BlockSpec((1, d), lambda i: (0, 0)),
                  pl.BlockSpec((1, d), lambda i: (0, 0)),
                  pl.BlockSpec((d, d), lambda i: (0, 0)),
                  pl.BlockSpec((1, d), lambda i: (0, 0))],
        out_specs=pl.BlockSpec((block, d), lambda i: (i, 0)),
        out_shape=jax.ShapeDtypeStruct((e, d), jnp.float32),
    )(g, a, stats, w1e, b1, gamma, beta, w2, b2)


def _tc_node(x, agg2, w1x, w1a, b1, gamma, beta, w2, b2):
    n, d = x.shape

    def body(x_ref, agg_ref, wx_ref, wa_ref, b1_ref, ga_ref, be_ref, w2_ref,
             b2_ref, out_ref):
        xv = x_ref[...]
        agg = (agg_ref[0] + agg_ref[1])[:n]
        h = (jnp.dot(xv, wx_ref[...], preferred_element_type=jnp.float32)
             + jnp.dot(agg, wa_ref[...], preferred_element_type=jnp.float32)
             + b1_ref[...])
        mu = jnp.mean(h, axis=0, keepdims=True)
        var = jnp.mean(jnp.square(h - mu), axis=0, keepdims=True)
        h = (h - mu) * lax.rsqrt(var + EPS) * ga_ref[...] + be_ref[...]
        h = jnp.maximum(h, 0.0)
        h = jnp.dot(h, w2_ref[...], preferred_element_type=jnp.float32) + b2_ref[...]
        out_ref[...] = jnp.maximum(h, 0.0) + xv

    return pl.pallas_call(
        body,
        out_shape=jax.ShapeDtypeStruct((n, d), jnp.float32),
    )(x, agg2, w1x, w1a, b1, gamma, beta, w2, b2)


def kernel(x, edge_index, edge_attr, params):
    n, d = x.shape
    e = edge_index.shape[1]
    nu_real = e // U
    upt = -(-nu_real // NW)
    pad = NW * upt * U - e

    def split3d(v):
        return jnp.pad(v, (0, pad)).reshape(NW, upt, U)

    src3d = split3d(edge_index[0])
    dst3d = split3d(edge_index[1])
    rpt = -(-((-(-n // NS))) // 8) * 8
    npad = NS * rpt
    block = 4000

    row = lambda v: v.reshape(1, d)
    for p in params:
        pe, pn = p["edge"], p["node"]
        w1i, w1j, w1e = pe["W1"][:d], pe["W1"][d:2 * d], pe["W1"][2 * d:]
        p_tab, q_tab = _tc_pre(x, w1i, w1j)
        g = _sc_gather(p_tab, q_tab, dst3d, src3d, nu_real)
        stats = _tc_stats(g, edge_attr, w1e, block)
        ue = _tc_ue(g, edge_attr, stats, w1e, row(pe["b1"]), row(pe["gamma"]),
                    row(pe["beta"]), pe["W2"], row(pe["b2"]), block)
        agg2 = _sc_scatter(ue, dst3d, nu_real, n, npad)
        x = _tc_node(x, agg2, pn["W1"][:d], pn["W1"][d:], row(pn["b1"]),
                     row(pn["gamma"]), row(pn["beta"]), pn["W2"], row(pn["b2"]))
        edge_attr = ue
    return (x, edge_index, edge_attr)


# speedup vs baseline: 1.5714x; 1.5714x over previous
import functools

import jax
import jax.numpy as jnp
from jax import lax
from jax.experimental import pallas as pl
from jax.experimental.pallas import tpu as pltpu
from jax.experimental.pallas import tpu_sc as plsc

EPS = 1e-5
LANES = 16
NC, NS = 2, 16
NW = NC * NS
U = 128


def _tc_pre(x, w1i, w1j):
    n, d = x.shape

    def body(x_ref, wi_ref, wj_ref, p_ref, q_ref):
        xv = x_ref[...]
        p_ref[...] = jnp.dot(xv, wi_ref[...], preferred_element_type=jnp.float32)
        q_ref[...] = jnp.dot(xv, wj_ref[...], preferred_element_type=jnp.float32)

    return pl.pallas_call(
        body,
        out_shape=(jax.ShapeDtypeStruct((n, d), jnp.float32),
                   jax.ShapeDtypeStruct((n, d), jnp.float32)),
    )(x, w1i, w1j)


def _sc_gather(p_tab, q_tab, dst3d, src3d, nu_real):
    _, upt, u = dst3d.shape
    n, d = p_tab.shape
    e = nu_real * u
    mesh = plsc.VectorSubcoreMesh(core_axis_name="c", subcore_axis_name="s")

    @functools.partial(
        pl.kernel,
        out_type=jax.ShapeDtypeStruct((e, d), jnp.float32),
        mesh=mesh,
        scratch_types=[
            pltpu.VMEM((upt, u), jnp.int32),
            pltpu.VMEM((upt, u), jnp.int32),
            pltpu.VMEM((2, u, d), jnp.float32),
            pltpu.VMEM((2, u, d), jnp.float32),
            pltpu.SemaphoreType.DMA,
            pltpu.SemaphoreType.DMA,
        ],
    )
    def k(p_hbm, q_hbm, d_hbm, s_hbm, out_hbm, idxd, idxs, bufa, bufb, gsem,
          wsem):
        w = lax.axis_index("s") * NC + lax.axis_index("c")
        u0 = w * upt
        nu = jnp.clip(nu_real - u0, 0, upt)
        pltpu.sync_copy(d_hbm.at[w], idxd)
        pltpu.sync_copy(s_hbm.at[w], idxs)

        def start_gather(j, slot):
            pltpu.async_copy(p_hbm.at[idxd.at[j]], bufa.at[slot], gsem)
            pltpu.async_copy(q_hbm.at[idxs.at[j]], bufb.at[slot], gsem)

        def drain(desc_src, desc_dst, sem):
            pltpu.make_async_copy(desc_src, desc_dst, sem).wait()

        @pl.when(nu > 0)
        def _():
            start_gather(0, 0)

        def unit(j, carry):
            slot = j % 2
            nslot = 1 - slot

            @pl.when((j + 1 < nu) & (j >= 1))
            def _():
                drain(bufa.at[nslot], out_hbm.at[pl.ds(0, u)], wsem)

            @pl.when(j + 1 < nu)
            def _():
                start_gather(j + 1, nslot)

            drain(p_hbm.at[idxd.at[j]], bufa.at[slot], gsem)
            drain(q_hbm.at[idxs.at[j]], bufb.at[slot], gsem)

            def add_rows(sslot):
                @functools.partial(plsc.parallel_loop, 0, u, unroll=4)
                def _(i):
                    for t in range(d // LANES):
                        sl = pl.ds(t * LANES, LANES)
                        bufa[sslot, i, sl] = (bufa[sslot, i, sl]
                                              + bufb[sslot, i, sl])

            @pl.when(slot == 0)
            def _():
                add_rows(0)

            @pl.when(slot == 1)
            def _():
                add_rows(1)
            pltpu.async_copy(bufa.at[slot], out_hbm.at[pl.ds((u0 + j) * u, u)],
                             wsem)
            return carry

        lax.fori_loop(0, nu, unit, 0)

        @pl.when(nu >= 2)
        def _():
            drain(bufa.at[0], out_hbm.at[pl.ds(0, u)], wsem)

        @pl.when(nu >= 1)
        def _():
            drain(bufa.at[0], out_hbm.at[pl.ds(0, u)], wsem)

    return k(p_tab, q_tab, dst3d, src3d)


def _sc_scatter(ue, dst3d, nu_real, n, npad):
    e, d = ue.shape
    _, upt, u = dst3d.shape
    rpt = npad // NS
    mesh = plsc.VectorSubcoreMesh(core_axis_name="c", subcore_axis_name="s")

    @functools.partial(
        pl.kernel,
        out_type=jax.ShapeDtypeStruct((NC, npad, d), jnp.float32),
        mesh=mesh,
        scratch_types=[
            pltpu.VMEM((upt, u), jnp.int32),
            pltpu.VMEM((2, u, d), jnp.float32),
            pltpu.VMEM_SHARED((npad, d), jnp.float32),
            pltpu.SemaphoreType.DMA,
            pltpu.SemaphoreType.DMA,
        ],
    )
    def k(ue_hbm, d_hbm, out_hbm, idxd, buf, agg_sh, rsem, ssem):
        c = lax.axis_index("c")
        s = lax.axis_index("s")
        w = s * NC + c
        u0 = w * upt
        nu = jnp.clip(nu_real - u0, 0, upt)

        def zrow(i, c2):
            for t in range(d // LANES):
                buf[0, i, pl.ds(t * LANES, LANES)] = jnp.zeros((LANES,),
                                                               jnp.float32)
            return c2

        lax.fori_loop(0, u, zrow, 0)
        nz, rem = divmod(rpt, u)
        for z in range(nz):
            pltpu.sync_copy(buf.at[0], agg_sh.at[pl.ds(s * rpt + z * u, u)])
        if rem:
            pltpu.sync_copy(buf.at[0, pl.ds(0, rem)],
                            agg_sh.at[pl.ds(s * rpt + nz * u, rem)])
        pltpu.sync_copy(d_hbm.at[w], idxd)
        plsc.subcore_barrier()

        def drain(desc_src, desc_dst, sem, add=False):
            pltpu.make_async_copy(desc_src, desc_dst, sem).wait()

        @pl.when(nu > 0)
        def _():
            pltpu.async_copy(ue_hbm.at[pl.ds(u0 * u, u)], buf.at[0], rsem)

        def unit(j, carry):
            slot = j % 2
            nslot = 1 - slot

            @pl.when((j + 1 < nu) & (j >= 1))
            def _():
                drain(buf.at[nslot], agg_sh.at[idxd.at[j]], ssem)

            @pl.when(j + 1 < nu)
            def _():
                pltpu.async_copy(ue_hbm.at[pl.ds((u0 + j + 1) * u, u)],
                                 buf.at[nslot], rsem)

            drain(ue_hbm.at[pl.ds(u0 * u, u)], buf.at[slot], rsem)
            pltpu.async_copy(buf.at[slot], agg_sh.at[idxd.at[j]], ssem,
                             add=True)
            return carry

        lax.fori_loop(0, nu, unit, 0)

        @pl.when(nu >= 2)
        def _():
            drain(buf.at[0], agg_sh.at[idxd.at[0]], ssem)

        @pl.when(nu >= 1)
        def _():
            drain(buf.at[0], agg_sh.at[idxd.at[0]], ssem)

        plsc.subcore_barrier()
        pltpu.sync_copy(agg_sh.at[pl.ds(s * rpt, rpt)],
                        out_hbm.at[c, pl.ds(s * rpt, rpt)])

    return k(ue, dst3d)


def _tc_stats(g, a, w1e, block):
    e, d = g.shape
    nb = e // block

    def body(g_ref, a_ref, w_ref, out_ref, acc_ref):
        i = pl.program_id(0)
        h = g_ref[...] + jnp.dot(a_ref[...], w_ref[...],
                                 preferred_element_type=jnp.float32)
        blk = jnp.concatenate(
            [jnp.sum(h, axis=0, keepdims=True),
             jnp.sum(h * h, axis=0, keepdims=True)], axis=0)

        @pl.when(i == 0)
        def _():
            acc_ref[...] = blk

        @pl.when(i > 0)
        def _():
            acc_ref[...] = acc_ref[...] + blk

        @pl.when(i == nb - 1)
        def _():
            out_ref[...] = acc_ref[...]

    return pl.pallas_call(
        body,
        grid=(nb,),
        in_specs=[pl.BlockSpec((block, d), lambda i: (i, 0)),
                  pl.BlockSpec((block, d), lambda i: (i, 0)),
                  pl.BlockSpec((d, d), lambda i: (0, 0))],
        out_specs=pl.BlockSpec((2, d), lambda i: (0, 0)),
        out_shape=jax.ShapeDtypeStruct((2, d), jnp.float32),
        scratch_shapes=[pltpu.VMEM((2, d), jnp.float32)],
    )(g, a, w1e)


def _tc_ue(g, a, stats, w1e, b1, gamma, beta, w2, b2, block):
    e, d = g.shape
    nb = e // block
    ef = float(e)

    def body(g_ref, a_ref, st_ref, w1_ref, b1_ref, ga_ref, be_ref, w2_ref,
             b2_ref, out_ref):
        mean = st_ref[0:1, :] / ef
        var = st_ref[1:2, :] / ef - mean * mean
        mu = mean + b1_ref[...]
        scale = lax.rsqrt(var + EPS) * ga_ref[...]
        av = a_ref[...]
        h = g_ref[...] + jnp.dot(av, w1_ref[...],
                                 preferred_element_type=jnp.float32) + b1_ref[...]
        h = (h - mu) * scale + be_ref[...]
        h = jnp.maximum(h, 0.0)
        h = jnp.dot(h, w2_ref[...], preferred_element_type=jnp.float32) + b2_ref[...]
        out_ref[...] = jnp.maximum(h, 0.0) + av

    return pl.pallas_call(
        body,
        grid=(nb,),
        in_specs=[pl.BlockSpec((block, d), lambda i: (i, 0)),
                  pl.BlockSpec((block, d), lambda i: (i, 0)),
                  pl.BlockSpec((2, d), lambda i: (0, 0)),
                  pl.BlockSpec((d, d), lambda i: (0, 0)),
                  pl.BlockSpec((1, d), lambda i: (0, 0)),
                  pl.BlockSpec((1, d), lambda i: (0, 0)),
                  pl.BlockSpec((1, d), lambda i: (0, 0)),
                  pl.BlockSpec((d, d), lambda i: (0, 0)),
                  pl.BlockSpec((1, d), lambda i: (0, 0))],
        out_specs=pl.BlockSpec((block, d), lambda i: (i, 0)),
        out_shape=jax.ShapeDtypeStruct((e, d), jnp.float32),
    )(g, a, stats, w1e, b1, gamma, beta, w2, b2)


def _tc_node(x, agg2, w1x, w1a, b1, gamma, beta, w2, b2):
    n, d = x.shape

    def body(x_ref, agg_ref, wx_ref, wa_ref, b1_ref, ga_ref, be_ref, w2_ref,
             b2_ref, out_ref):
        xv = x_ref[...]
        agg = (agg_ref[0] + agg_ref[1])[:n]
        h = (jnp.dot(xv, wx_ref[...], preferred_element_type=jnp.float32)
             + jnp.dot(agg, wa_ref[...], preferred_element_type=jnp.float32)
             + b1_ref[...])
        mu = jnp.mean(h, axis=0, keepdims=True)
        var = jnp.mean(jnp.square(h - mu), axis=0, keepdims=True)
        h = (h - mu) * lax.rsqrt(var + EPS) * ga_ref[...] + be_ref[...]
        h = jnp.maximum(h, 0.0)
        h = jnp.dot(h, w2_ref[...], preferred_element_type=jnp.float32) + b2_ref[...]
        out_ref[...] = jnp.maximum(h, 0.0) + xv

    return pl.pallas_call(
        body,
        out_shape=jax.ShapeDtypeStruct((n, d), jnp.float32),
    )(x, agg2, w1x, w1a, b1, gamma, beta, w2, b2)


def kernel(x, edge_index, edge_attr, params):
    n, d = x.shape
    e = edge_index.shape[1]
    nu_real = e // U
    upt = -(-nu_real // NW)
    pad = NW * upt * U - e

    def split3d(v):
        return jnp.pad(v, (0, pad)).reshape(NW, upt, U)

    src3d = split3d(edge_index[0])
    dst3d = split3d(edge_index[1])
    rpt = -(-((-(-n // NS))) // 8) * 8
    npad = NS * rpt
    block = 4000

    row = lambda v: v.reshape(1, d)
    for p in params:
        pe, pn = p["edge"], p["node"]
        w1i, w1j, w1e = pe["W1"][:d], pe["W1"][d:2 * d], pe["W1"][2 * d:]
        p_tab, q_tab = _tc_pre(x, w1i, w1j)
        g = _sc_gather(p_tab, q_tab, dst3d, src3d, nu_real)
        stats = _tc_stats(g, edge_attr, w1e, block)
        ue = _tc_ue(g, edge_attr, stats, w1e, row(pe["b1"]), row(pe["gamma"]),
                    row(pe["beta"]), pe["W2"], row(pe["b2"]), block)
        agg2 = _sc_scatter(ue, dst3d, nu_real, n, npad)
        x = _tc_node(x, agg2, pn["W1"][:d], pn["W1"][d:], row(pn["b1"]),
                     row(pn["gamma"]), row(pn["beta"]), pn["W2"], row(pn["b2"]))
        edge_attr = ue
    return (x, edge_index, edge_attr)
